# baseline (speedup 1.0000x reference)
import jax
import jax.numpy as jnp
from jax.experimental import pallas as pl
from jax.experimental.pallas import tpu as pltpu

_B = 2048


def _enqueue_kernel(ptr_ref, fq_ref, lq_ref, keys_ref, lab_ref, fq_out, lq_out):
    K = fq_ref.shape[0] * pl.num_programs(0)
    BS = keys_ref.shape[0] - 2 * _B
    r0 = pl.program_id(0) * _B
    ptr = ptr_ref[0]

    j0 = (r0 - ptr) & (K - 1)
    j0 = jnp.where(j0 >= K - _B, j0 - K, j0)
    s = jnp.minimum(j0, BS) + _B

    rows = r0 + jax.lax.broadcasted_iota(jnp.int32, (_B, 1), 0)
    in_window = ((rows - ptr) & (K - 1)) < BS

    cand = keys_ref[pl.ds(s, _B), :]
    fq_out[...] = jnp.where(in_window, cand, fq_ref[...])
    cand_l = lab_ref[pl.ds(s, _B), :]
    lq_out[...] = jnp.where(in_window, cand_l, lq_ref[...])


def kernel(feature_queue, label_queue, ptr, keys, labels):
    K, D = feature_queue.shape
    BS = keys.shape[0]
    keys_pad = jnp.pad(keys, ((_B, _B), (0, 0)))
    lab_pad = jnp.pad(labels.astype(label_queue.dtype), (_B, _B))[:, None]
    lq2 = label_queue[:, None]
    ptr1 = jnp.reshape(ptr, (1,)).astype(jnp.int32)

    new_fq, new_lq = pl.pallas_call(
        _enqueue_kernel,
        grid=(K // _B,),
        in_specs=[
            pl.BlockSpec(memory_space=pltpu.SMEM),
            pl.BlockSpec((_B, D), lambda i: (i, 0)),
            pl.BlockSpec((_B, 1), lambda i: (i, 0)),
            pl.BlockSpec((BS + 2 * _B, D), lambda i: (0, 0)),
            pl.BlockSpec((BS + 2 * _B, 1), lambda i: (0, 0)),
        ],
        out_specs=[
            pl.BlockSpec((_B, D), lambda i: (i, 0)),
            pl.BlockSpec((_B, 1), lambda i: (i, 0)),
        ],
        out_shape=[
            jax.ShapeDtypeStruct((K, D), feature_queue.dtype),
            jax.ShapeDtypeStruct((K, 1), label_queue.dtype),
        ],
    )(ptr1, feature_queue, lq2, keys_pad, lab_pad)

    new_ptr = ((ptr + BS) % K).astype(ptr.dtype)
    return new_fq, new_lq[:, 0], new_ptr
